# Initial kernel scaffold; baseline (speedup 1.0000x reference)
import jax
import jax.numpy as jnp
from jax.experimental import pallas as pl


def kernel(x, u):
    raise NotImplementedError("write your pallas kernel here")


# speedup vs baseline: 6.8133x; 6.8133x over previous
import functools

import jax
import jax.numpy as jnp
from jax import lax
from jax.experimental import pallas as pl
from jax.experimental.pallas import tpu as pltpu
from jax.experimental.pallas import tpu_sc as plsc

B, C, M, N = 4, 8, 512, 512
PLANE = M * N
CH_ROWS = 8
CHUNK_PX = CH_ROWS * N
NVEC = CHUNK_PX // 16
NCHUNK = M // CH_ROWS
SCAT = 4 * CHUNK_PX


def _tile_body(xf, uf, out, xbuf, ubuf, idxb, valb, acc):
    sc = lax.axis_index("c")
    sub = lax.axis_index("s")
    p = sub >> 3
    ch = sub & 7
    b = 2 * sc + p
    plane = b * C + ch
    base_p = p * PLANE

    lane = jnp.arange(16, dtype=jnp.int32)
    lane2 = lane * 2

    def _z(i, _):
        valb[pl.ds(i * 16, 16)] = jnp.zeros((16,), jnp.float32)
        return 0
    lax.fori_loop(0, 1024, _z, 0)
    pltpu.sync_copy(valb, acc.at[pl.ds(sub * 32768, 16384)])
    pltpu.sync_copy(valb, acc.at[pl.ds(sub * 32768 + 16384, 16384)])
    plsc.subcore_barrier()

    xoff = plane * PLANE

    def chunk_body(cidx, _):
        r0 = cidx * CH_ROWS
        pltpu.sync_copy(xf.at[pl.ds(xoff + r0 * N, CHUNK_PX)], xbuf)
        pltpu.sync_copy(uf.at[pl.ds((xoff + r0 * N) * 2, 2 * CHUNK_PX)], ubuf)

        def g_body(g, _):
            r = r0 + (g >> 5)
            colb = (g & 31) * 16
            ub = g * 32 + lane2
            dx = plsc.load_gather(ubuf, [ub])
            dy = plsc.load_gather(ubuf, [ub + 1])
            px = dx + (colb + lane).astype(jnp.float32)
            py = dy + r.astype(jnp.float32)
            xt = px.astype(jnp.int32)
            xtf = xt.astype(jnp.float32)
            x0 = jnp.where(xtf > px, xt - 1, xt)
            yt = py.astype(jnp.int32)
            ytf = yt.astype(jnp.float32)
            y0 = jnp.where(ytf > py, yt - 1, yt)
            wx = px - x0.astype(jnp.float32)
            wy = py - y0.astype(jnp.float32)
            xv = xbuf[pl.ds(g * 16, 16)]

            x1 = x0 + 1
            y1 = y0 + 1
            x0c = jnp.clip(x0, 0, N - 1)
            x1c = jnp.clip(x1, 0, N - 1)
            y0c = jnp.clip(y0, 0, M - 1)
            y1c = jnp.clip(y1, 0, M - 1)
            vx0 = x0c == x0
            vx1 = x1c == x1
            vy0 = y0c == y0
            vy1 = y1c == y1
            fx0 = 1.0 - wx
            fy0 = 1.0 - wy
            xb0 = x0c + base_p
            xb1 = x1c + base_p
            ym0 = y0c * N
            ym1 = y1c * N

            o = g * 16
            czero = jnp.zeros((16,), jnp.float32)
            w = jnp.where(vx0 & vy0, fx0 * fy0, czero)
            idxb[pl.ds(o, 16)] = ym0 + xb0
            valb[pl.ds(o, 16)] = w * xv
            w = jnp.where(vx1 & vy0, wx * fy0, czero)
            idxb[pl.ds(o + CHUNK_PX, 16)] = ym0 + xb1
            valb[pl.ds(o + CHUNK_PX, 16)] = w * xv
            w = jnp.where(vx0 & vy1, fx0 * wy, czero)
            idxb[pl.ds(o + 2 * CHUNK_PX, 16)] = ym1 + xb0
            valb[pl.ds(o + 2 * CHUNK_PX, 16)] = w * xv
            w = jnp.where(vx1 & vy1, wx * wy, czero)
            idxb[pl.ds(o + 3 * CHUNK_PX, 16)] = ym1 + xb1
            valb[pl.ds(o + 3 * CHUNK_PX, 16)] = w * xv
            return 0

        lax.fori_loop(0, NVEC, g_body, 0)
        pltpu.sync_copy(valb, acc.at[idxb], add=True)
        return 0

    lax.fori_loop(0, NCHUNK, chunk_body, 0)
    plsc.subcore_barrier()

    off = ch * 32768
    pltpu.sync_copy(acc.at[pl.ds(base_p + off, 32768)],
                    out.at[pl.ds(b * PLANE + off, 32768)])


@jax.jit
def _warp_adjoint_sc(xf, uf):
    mesh = plsc.VectorSubcoreMesh(core_axis_name="c", subcore_axis_name="s")
    return pl.kernel(
        _tile_body,
        out_type=jax.ShapeDtypeStruct((B * PLANE,), jnp.float32),
        mesh=mesh,
        compiler_params=pltpu.CompilerParams(needs_layout_passes=False),
        scratch_types=[
            pltpu.VMEM((CHUNK_PX,), jnp.float32),
            pltpu.VMEM((2 * CHUNK_PX,), jnp.float32),
            pltpu.VMEM((SCAT,), jnp.int32),
            pltpu.VMEM((SCAT,), jnp.float32),
            pltpu.VMEM_SHARED((2 * PLANE,), jnp.float32),
        ],
    )(xf, uf)


def kernel(x, u):
    xf = jnp.reshape(x, (-1,))
    uf = jnp.reshape(u, (-1,))
    out = _warp_adjoint_sc(xf, uf)
    return jnp.reshape(out, (B, M, N))


# speedup vs baseline: 100.5241x; 14.7541x over previous
import functools

import jax
import jax.numpy as jnp
from jax import lax
from jax.experimental import pallas as pl
from jax.experimental.pallas import tpu as pltpu
from jax.experimental.pallas import tpu_sc as plsc

B, C, M, N = 4, 8, 512, 512
PLANE = M * N
CH_ROWS = 8
CHUNK_PX = CH_ROWS * N
NVEC = CHUNK_PX // 16
NCHUNK = M // CH_ROWS
SCAT = 4 * CHUNK_PX


def _tile_body(xf, uxf, uyf, out, xbuf, uxbuf, uybuf, idxb, valb, acc):
    sc = lax.axis_index("c")
    sub = lax.axis_index("s")
    p = sub >> 3
    ch = sub & 7
    b = 2 * sc + p
    plane = b * C + ch
    base_p = p * PLANE

    lane = jnp.arange(16, dtype=jnp.int32)

    def _z(i, _):
        valb[pl.ds(i * 16, 16)] = jnp.zeros((16,), jnp.float32)
        return 0
    lax.fori_loop(0, 1024, _z, 0)
    pltpu.sync_copy(valb, acc.at[pl.ds(sub * 32768, 16384)])
    pltpu.sync_copy(valb, acc.at[pl.ds(sub * 32768 + 16384, 16384)])
    plsc.subcore_barrier()

    xoff = plane * PLANE

    def chunk_body(cidx, _):
        r0 = cidx * CH_ROWS
        pltpu.sync_copy(xf.at[pl.ds(xoff + r0 * N, CHUNK_PX)], xbuf)
        pltpu.sync_copy(uxf.at[pl.ds(xoff + r0 * N, CHUNK_PX)], uxbuf)
        pltpu.sync_copy(uyf.at[pl.ds(xoff + r0 * N, CHUNK_PX)], uybuf)

        def g_body(g, _):
            r = r0 + (g >> 5)
            colb = (g & 31) * 16
            dx = uxbuf[pl.ds(g * 16, 16)]
            dy = uybuf[pl.ds(g * 16, 16)]
            px = dx + (colb + lane).astype(jnp.float32)
            py = dy + r.astype(jnp.float32)
            xt = px.astype(jnp.int32)
            xtf = xt.astype(jnp.float32)
            x0 = jnp.where(xtf > px, xt - 1, xt)
            yt = py.astype(jnp.int32)
            ytf = yt.astype(jnp.float32)
            y0 = jnp.where(ytf > py, yt - 1, yt)
            wx = px - x0.astype(jnp.float32)
            wy = py - y0.astype(jnp.float32)
            xv = xbuf[pl.ds(g * 16, 16)]

            x1 = x0 + 1
            y1 = y0 + 1
            x0c = jnp.clip(x0, 0, N - 1)
            x1c = jnp.clip(x1, 0, N - 1)
            y0c = jnp.clip(y0, 0, M - 1)
            y1c = jnp.clip(y1, 0, M - 1)
            vx0 = x0c == x0
            vx1 = x1c == x1
            vy0 = y0c == y0
            vy1 = y1c == y1
            fx0 = 1.0 - wx
            fy0 = 1.0 - wy
            xb0 = x0c + base_p
            xb1 = x1c + base_p
            ym0 = y0c * N
            ym1 = y1c * N

            o = g * 16
            czero = jnp.zeros((16,), jnp.float32)
            w = jnp.where(vx0 & vy0, fx0 * fy0, czero)
            idxb[pl.ds(o, 16)] = ym0 + xb0
            valb[pl.ds(o, 16)] = w * xv
            w = jnp.where(vx1 & vy0, wx * fy0, czero)
            idxb[pl.ds(o + CHUNK_PX, 16)] = ym0 + xb1
            valb[pl.ds(o + CHUNK_PX, 16)] = w * xv
            w = jnp.where(vx0 & vy1, fx0 * wy, czero)
            idxb[pl.ds(o + 2 * CHUNK_PX, 16)] = ym1 + xb0
            valb[pl.ds(o + 2 * CHUNK_PX, 16)] = w * xv
            w = jnp.where(vx1 & vy1, wx * wy, czero)
            idxb[pl.ds(o + 3 * CHUNK_PX, 16)] = ym1 + xb1
            valb[pl.ds(o + 3 * CHUNK_PX, 16)] = w * xv
            return 0

        lax.fori_loop(0, NVEC, g_body, 0)
        pltpu.sync_copy(valb, acc.at[idxb], add=True)
        return 0

    lax.fori_loop(0, NCHUNK, chunk_body, 0)
    plsc.subcore_barrier()

    off = ch * 32768
    pltpu.sync_copy(acc.at[pl.ds(base_p + off, 32768)],
                    out.at[pl.ds(b * PLANE + off, 32768)])


@jax.jit
def _warp_adjoint_sc(xf, uxf, uyf):
    mesh = plsc.VectorSubcoreMesh(core_axis_name="c", subcore_axis_name="s")
    return pl.kernel(
        _tile_body,
        out_type=jax.ShapeDtypeStruct((B * PLANE,), jnp.float32),
        mesh=mesh,
        compiler_params=pltpu.CompilerParams(needs_layout_passes=False),
        scratch_types=[
            pltpu.VMEM((CHUNK_PX,), jnp.float32),
            pltpu.VMEM((CHUNK_PX,), jnp.float32),
            pltpu.VMEM((CHUNK_PX,), jnp.float32),
            pltpu.VMEM((SCAT,), jnp.int32),
            pltpu.VMEM((SCAT,), jnp.float32),
            pltpu.VMEM_SHARED((2 * PLANE,), jnp.float32),
        ],
    )(xf, uxf, uyf)


def kernel(x, u):
    xf = jnp.reshape(x, (-1,))
    uxf = jnp.reshape(u[..., 0], (-1,))
    uyf = jnp.reshape(u[..., 1], (-1,))
    out = _warp_adjoint_sc(xf, uxf, uyf)
    return jnp.reshape(out, (B, M, N))
